# baseline (speedup 1.0000x reference)
import functools

import jax
import jax.numpy as jnp
from jax import lax
from jax.experimental import pallas as pl
from jax.experimental.pallas import tpu as pltpu
from jax.experimental.pallas import tpu_sc as plsc

NC = 2
NS = 16
NW = NC * NS
LANES = 16
C = 128
CV = C // LANES

f32 = jnp.float32


def _dot(a, b):
    return jnp.dot(a, b, preferred_element_type=f32)


def _stage_a_body(x_ref, pos_ref, W_in_ref, b_in_ref, W_lin_ref, W_src_ref,
                  W_dst_ref, pW1_ref, aW1_ref,
                  p1_ref, ad1_ref, as1_ref, hl_ref):
    h = jax.nn.relu(_dot(x_ref[...], W_in_ref[...]) + b_in_ref[...])
    p1_ref[...] = _dot(pos_ref[...], pW1_ref[...])
    ad1_ref[...] = _dot(h, _dot(W_dst_ref[...], aW1_ref[...]))
    as1_ref[...] = _dot(h, _dot(W_src_ref[...], aW1_ref[...]))
    hl_ref[...] = _dot(h, W_lin_ref[...])


def _stage_a(x, pos, W_in, b_in, W_lin, W_src, W_dst, pW1, aW1, n_blk):
    n = x.shape[0]
    grid = (n // n_blk,)
    row_spec = pl.BlockSpec((n_blk, C), lambda i: (i, 0))
    w_spec = pl.BlockSpec((C, C), lambda i: (0, 0))
    b_spec = pl.BlockSpec((1, C), lambda i: (0, 0))
    out_sh = jax.ShapeDtypeStruct((n, C), f32)
    return pl.pallas_call(
        _stage_a_body,
        grid=grid,
        in_specs=[row_spec, row_spec, w_spec, b_spec, w_spec, w_spec, w_spec,
                  w_spec, w_spec],
        out_specs=[row_spec, row_spec, row_spec, row_spec],
        out_shape=[out_sh, out_sh, out_sh, out_sh],
        compiler_params=pltpu.CompilerParams(
            dimension_semantics=("arbitrary",)),
    )(x, pos, W_in, b_in.reshape(1, C), W_lin, W_src, W_dst, pW1, aW1)


def _stage_b(p1, ad1, as1, sd, K):
    e = sd.shape[0] // 2
    n = p1.shape[0]
    ew = e // NW
    nch = ew // K
    npair = nch // 2
    mesh = plsc.VectorSubcoreMesh(core_axis_name="c", subcore_axis_name="s")
    out_sh = jax.ShapeDtypeStruct((e, C), f32)

    slot_t = [
        pltpu.VMEM((2 * K,), jnp.int32),
        pltpu.VMEM((K, C), f32),
        pltpu.VMEM((K, C), f32),
        pltpu.VMEM((K, C), f32),
        pltpu.VMEM((K, C), f32),
        pltpu.SemaphoreType.DMA,
        pltpu.SemaphoreType.DMA,
    ]

    @functools.partial(
        pl.kernel,
        out_type=[out_sh, out_sh],
        mesh=mesh,
        scratch_types=slot_t + slot_t,
    )
    def kern(p1_h, ad1_h, as1_h, sd_h, g1_h, w_h,
             sdA, bd1A, bs1A, bd2A, bs2A, gsA, wsA,
             sdB, bd1B, bs1B, bd2B, bs2B, gsB, wsB):
        cid = lax.axis_index("c")
        sid = lax.axis_index("s")
        wid = cid * NS + sid
        base = wid * ew

        slots = ((sdA, bd1A, bs1A, bd2A, bs2A, gsA, wsA),
                 (sdB, bd1B, bs1B, bd2B, bs2B, gsB, wsB))

        def issue(s, off):
            sdx, bd1, bs1, bd2, bs2, gs, _ = slots[s]
            pltpu.sync_copy(sd_h.at[pl.ds(2 * off, 2 * K)], sdx)
            six = sdx.at[pl.ds(0, K)]
            dix = sdx.at[pl.ds(K, K)]
            pltpu.async_copy(p1_h.at[dix], bd1, gs)
            pltpu.async_copy(p1_h.at[six], bs1, gs)
            pltpu.async_copy(ad1_h.at[dix], bd2, gs)
            pltpu.async_copy(as1_h.at[six], bs2, gs)

        def drain_writes(s):
            _, bd1, _, bd2, _, _, ws = slots[s]
            pltpu.make_async_copy(bd1, g1_h.at[pl.ds(0, K)], ws).wait()
            pltpu.make_async_copy(bd2, w_h.at[pl.ds(0, K)], ws).wait()

        def process(s, off):
            sdx, bd1, bs1, bd2, bs2, gs, ws = slots[s]
            six = sdx.at[pl.ds(0, K)]
            dix = sdx.at[pl.ds(K, K)]
            pltpu.make_async_copy(p1_h.at[dix], bd1, gs).wait()
            pltpu.make_async_copy(p1_h.at[six], bs1, gs).wait()
            pltpu.make_async_copy(ad1_h.at[dix], bd2, gs).wait()
            pltpu.make_async_copy(as1_h.at[six], bs2, gs).wait()

            def row(r, carry2):
                for j in range(CV):
                    sl = pl.ds(j * LANES, LANES)
                    bd1[r, sl] = bd1[r, sl] - bs1[r, sl]
                    bd2[r, sl] = bd2[r, sl] - bs2[r, sl]
                return carry2

            lax.fori_loop(0, K, row, 0)
            pltpu.async_copy(bd1, g1_h.at[pl.ds(off, K)], ws)
            pltpu.async_copy(bd2, w_h.at[pl.ds(off, K)], ws)

        issue(0, base)

        def pair(g, carry):
            offA = base + (2 * g) * K
            offB = offA + K

            @pl.when(g > 0)
            def _():
                drain_writes(1)

            issue(1, offB)
            process(0, offA)

            @pl.when(g < npair - 1)
            def _():
                drain_writes(0)
                issue(0, offA + 2 * K)

            process(1, offB)
            return carry

        lax.fori_loop(0, npair, pair, 0)
        if nch % 2 == 1:
            off = base + (nch - 1) * K
            drain_writes(0)
            issue(0, off)
            process(0, off)
        drain_writes(0)
        drain_writes(1)

    return kern(p1, ad1, as1, sd)


def _stage_c_body(g1_ref, w_ref, pW2_ref, aW1_ref, aW2_ref,
                  pb1_ref, pb2_ref, ab1_ref, ab2_ref, ex_ref, exd_ref):
    u = jax.nn.relu(g1_ref[...] + pb1_ref[...])
    delta = jax.nn.relu(_dot(u, pW2_ref[...]) + pb2_ref[...])
    v = jax.nn.relu(w_ref[...] + _dot(delta, aW1_ref[...]) + ab1_ref[...])
    alpha = jax.nn.relu(_dot(v, aW2_ref[...]) + ab2_ref[...])
    ex = jnp.exp(jnp.minimum(alpha, 80.0))
    ex_ref[...] = ex
    exd_ref[...] = ex * delta


def _stage_c(g1, w, pW2, aW1, aW2, pb1, pb2, ab1, ab2, e_blk):
    e = g1.shape[0]
    grid = (e // e_blk,)
    row_spec = pl.BlockSpec((e_blk, C), lambda i: (i, 0))
    w_spec = pl.BlockSpec((C, C), lambda i: (0, 0))
    b_spec = pl.BlockSpec((1, C), lambda i: (0, 0))
    out_sh = jax.ShapeDtypeStruct((e, C), f32)
    return pl.pallas_call(
        _stage_c_body,
        grid=grid,
        in_specs=[row_spec, row_spec, w_spec, w_spec, w_spec,
                  b_spec, b_spec, b_spec, b_spec],
        out_specs=[row_spec, row_spec],
        out_shape=[out_sh, out_sh],
        compiler_params=pltpu.CompilerParams(
            dimension_semantics=("arbitrary",)),
    )(g1, w, pW2, aW1, aW2, pb1.reshape(1, C), pb2.reshape(1, C),
      ab1.reshape(1, C), ab2.reshape(1, C))


def _stage_d_den(ex, dst, n, K):
    e = ex.shape[0]
    ew = e // NW
    nch = ew // K
    npair = nch // 2
    npad = ((n + NS * K - 1) // (NS * K)) * (NS * K)
    zr = npad // NS
    nzc = zr // K

    mesh = plsc.VectorSubcoreMesh(core_axis_name="c", subcore_axis_name="s")

    slot_t = [
        pltpu.VMEM((K,), jnp.int32),
        pltpu.VMEM((K, C), f32),
        pltpu.SemaphoreType.DMA,
        pltpu.SemaphoreType.DMA,
    ]

    @functools.partial(
        pl.kernel,
        out_type=jax.ShapeDtypeStruct((NC, npad, C), f32),
        mesh=mesh,
        scratch_types=slot_t + slot_t + [pltpu.VMEM_SHARED((npad, C), f32)],
    )
    def kern(ex_h, dst_h, den_h,
             dixA, exbA, gsA, ssA, dixB, exbB, gsB, ssB, den_sh):
        cid = lax.axis_index("c")
        sid = lax.axis_index("s")
        wid = cid * NS + sid
        base = wid * ew
        slots = ((dixA, exbA, gsA, ssA), (dixB, exbB, gsB, ssB))

        def zrow(r, carry):
            for j in range(CV):
                exbA[r, pl.ds(j * LANES, LANES)] = jnp.zeros((LANES,), f32)
            return carry

        lax.fori_loop(0, K, zrow, 0)

        def zcopy(k, carry):
            pltpu.sync_copy(exbA, den_sh.at[pl.ds(sid * zr + k * K, K)])
            return carry

        lax.fori_loop(0, nzc, zcopy, 0)
        plsc.subcore_barrier()

        def issue(s, off):
            dix, exb, gs, _ = slots[s]
            pltpu.sync_copy(dst_h.at[pl.ds(off, K)], dix)
            pltpu.async_copy(ex_h.at[pl.ds(off, K)], exb, gs)

        def drain_scatter(s):
            dix, exb, _, ss = slots[s]
            pltpu.make_async_copy(exb, den_sh.at[dix], ss).wait()

        def process(s, off):
            dix, exb, gs, ss = slots[s]
            pltpu.make_async_copy(ex_h.at[pl.ds(off, K)], exb, gs).wait()
            pltpu.async_copy(exb, den_sh.at[dix], ss, add=True)

        issue(0, base)

        def pair(g, carry):
            offA = base + (2 * g) * K
            offB = offA + K

            @pl.when(g > 0)
            def _():
                drain_scatter(1)

            issue(1, offB)
            process(0, offA)

            @pl.when(g < npair - 1)
            def _():
                drain_scatter(0)
                issue(0, offA + 2 * K)

            process(1, offB)
            return carry

        lax.fori_loop(0, npair, pair, 0)
        if nch % 2 == 1:
            off = base + (nch - 1) * K
            drain_scatter(0)
            issue(0, off)
            process(0, off)
        drain_scatter(0)
        drain_scatter(1)
        plsc.subcore_barrier()

        def wcopy(k, carry):
            pltpu.sync_copy(den_sh.at[pl.ds(sid * zr + k * K, K)], exbA)
            pltpu.sync_copy(exbA, den_h.at[cid, pl.ds(sid * zr + k * K, K)])
            return carry

        lax.fori_loop(0, nzc, wcopy, 0)

    return kern(ex, dst)


def _stage_d_num(ex, exd, hl, src, dst, n, K):
    e = ex.shape[0]
    ew = e // NW
    nch = ew // K
    npair = nch // 2
    npad = ((n + NS * K - 1) // (NS * K)) * (NS * K)
    zr = npad // NS
    nzc = zr // K

    mesh = plsc.VectorSubcoreMesh(core_axis_name="c", subcore_axis_name="s")

    slot_t = [
        pltpu.VMEM((K,), jnp.int32),
        pltpu.VMEM((K,), jnp.int32),
        pltpu.VMEM((K, C), f32),
        pltpu.VMEM((K, C), f32),
        pltpu.VMEM((K, C), f32),
        pltpu.SemaphoreType.DMA,
        pltpu.SemaphoreType.DMA,
    ]

    @functools.partial(
        pl.kernel,
        out_type=jax.ShapeDtypeStruct((NC, npad, C), f32),
        mesh=mesh,
        scratch_types=slot_t + slot_t + [pltpu.VMEM_SHARED((npad, C), f32)],
    )
    def kern(ex_h, exd_h, hl_h, src_h, dst_h, num_h,
             sixA, dixA, exbA, exdbA, hlbA, gsA, ssA,
             sixB, dixB, exbB, exdbB, hlbB, gsB, ssB, num_sh):
        cid = lax.axis_index("c")
        sid = lax.axis_index("s")
        wid = cid * NS + sid
        base = wid * ew
        slots = ((sixA, dixA, exbA, exdbA, hlbA, gsA, ssA),
                 (sixB, dixB, exbB, exdbB, hlbB, gsB, ssB))

        def zrow(r, carry):
            for j in range(CV):
                exbA[r, pl.ds(j * LANES, LANES)] = jnp.zeros((LANES,), f32)
            return carry

        lax.fori_loop(0, K, zrow, 0)

        def zcopy(k, carry):
            pltpu.sync_copy(exbA, num_sh.at[pl.ds(sid * zr + k * K, K)])
            return carry

        lax.fori_loop(0, nzc, zcopy, 0)
        plsc.subcore_barrier()

        def issue(s, off):
            six, dix, exb, exdb, hlb, gs, _ = slots[s]
            pltpu.sync_copy(src_h.at[pl.ds(off, K)], six)
            pltpu.sync_copy(dst_h.at[pl.ds(off, K)], dix)
            pltpu.async_copy(hl_h.at[six], hlb, gs)
            pltpu.async_copy(ex_h.at[pl.ds(off, K)], exb, gs)
            pltpu.async_copy(exd_h.at[pl.ds(off, K)], exdb, gs)

        def drain_scatter(s):
            _, dix, _, exdb, _, _, ss = slots[s]
            pltpu.make_async_copy(exdb, num_sh.at[dix], ss).wait()

        def process(s, off):
            six, dix, exb, exdb, hlb, gs, ss = slots[s]
            pltpu.make_async_copy(hl_h.at[six], hlb, gs).wait()
            pltpu.make_async_copy(ex_h.at[pl.ds(off, K)], exb, gs).wait()
            pltpu.make_async_copy(exd_h.at[pl.ds(off, K)], exdb, gs).wait()

            def row(r, carry2):
                for j in range(CV):
                    sl = pl.ds(j * LANES, LANES)
                    exdb[r, sl] = exdb[r, sl] + exb[r, sl] * hlb[r, sl]
                return carry2

            lax.fori_loop(0, K, row, 0)
            pltpu.async_copy(exdb, num_sh.at[dix], ss, add=True)

        issue(0, base)

        def pair(g, carry):
            offA = base + (2 * g) * K
            offB = offA + K

            @pl.when(g > 0)
            def _():
                drain_scatter(1)

            issue(1, offB)
            process(0, offA)

            @pl.when(g < npair - 1)
            def _():
                drain_scatter(0)
                issue(0, offA + 2 * K)

            process(1, offB)
            return carry

        lax.fori_loop(0, npair, pair, 0)
        if nch % 2 == 1:
            off = base + (nch - 1) * K
            drain_scatter(0)
            issue(0, off)
            process(0, off)
        drain_scatter(0)
        drain_scatter(1)
        plsc.subcore_barrier()

        def wcopy(k, carry):
            pltpu.sync_copy(num_sh.at[pl.ds(sid * zr + k * K, K)], exbA)
            pltpu.sync_copy(exbA, num_h.at[cid, pl.ds(sid * zr + k * K, K)])
            return carry

        lax.fori_loop(0, nzc, wcopy, 0)

    return kern(ex, exd, hl, src, dst)


def _stage_e_body(num0_ref, num1_ref, den0_ref, den1_ref,
                  W_out_ref, b_out_ref, out_ref):
    num = (num0_ref[0] + num0_ref[1]) + (num1_ref[0] + num1_ref[1])
    den = (den0_ref[0] + den0_ref[1]) + (den1_ref[0] + den1_ref[1])
    r = num / (den + 1e-16)
    out_ref[...] = jax.nn.relu(_dot(r, W_out_ref[...]) + b_out_ref[...])


def _stage_e(num0, num1, den0, den1, W_out, b_out, n, n_blk):
    grid = (n // n_blk,)
    part_spec = pl.BlockSpec((NC, n_blk, C), lambda i: (0, i, 0))
    w_spec = pl.BlockSpec((C, C), lambda i: (0, 0))
    b_spec = pl.BlockSpec((1, C), lambda i: (0, 0))
    return pl.pallas_call(
        _stage_e_body,
        grid=grid,
        in_specs=[part_spec, part_spec, part_spec, part_spec, w_spec, b_spec],
        out_specs=pl.BlockSpec((n_blk, C), lambda i: (i, 0)),
        out_shape=jax.ShapeDtypeStruct((n, C), f32),
        compiler_params=pltpu.CompilerParams(
            dimension_semantics=("arbitrary",)),
    )(num0, num1, den0, den1, W_out, b_out.reshape(1, C))


def kernel(x, pos, edge_index, W_in, b_in, W_out, b_out, W_lin, W_src, W_dst,
           pW1, pb1, pW2, pb2, aW1, ab1, aW2, ab2):
    n = x.shape[0]
    src = edge_index[0]
    dst = edge_index[1]

    K = 40
    n_blk = 1000
    e_blk = 2000
    e = src.shape[0]
    eh = e // 2

    p1, ad1, as1, hl = _stage_a(x, pos, W_in, b_in, W_lin, W_src, W_dst,
                                pW1, aW1, n_blk)
    halves = []
    for h in range(2):
        s_h = src[h * eh:(h + 1) * eh]
        d_h = dst[h * eh:(h + 1) * eh]
        sd_h = jnp.stack([s_h.reshape(-1, K), d_h.reshape(-1, K)],
                         axis=1).reshape(-1)
        halves.append((s_h, d_h, sd_h))

    g1_0, w_0 = _stage_b(p1, ad1, as1, halves[0][2], K)
    g1_1, w_1 = _stage_b(p1, ad1, as1, halves[1][2], K)
    ex0, exd0 = _stage_c(g1_0, w_0, pW2, aW1, aW2, pb1, pb2, ab1, ab2, e_blk)
    den0 = _stage_d_den(ex0, halves[0][1], n, K)
    num0 = _stage_d_num(ex0, exd0, hl, halves[0][0], halves[0][1], n, K)
    ex1, exd1 = _stage_c(g1_1, w_1, pW2, aW1, aW2, pb1, pb2, ab1, ab2, e_blk)
    den1 = _stage_d_den(ex1, halves[1][1], n, K)
    num1 = _stage_d_num(ex1, exd1, hl, halves[1][0], halves[1][1], n, K)
    return _stage_e(num0, num1, den0, den1, W_out, b_out, n, n_blk)


# speedup vs baseline: 1.1106x; 1.1106x over previous
import functools

import jax
import jax.numpy as jnp
from jax import lax
from jax.experimental import pallas as pl
from jax.experimental.pallas import tpu as pltpu
from jax.experimental.pallas import tpu_sc as plsc

NC = 2
NS = 16
NW = NC * NS
LANES = 16
C = 128
CV = C // LANES

f32 = jnp.float32


def _dot(a, b):
    return jnp.dot(a, b, preferred_element_type=f32)


def _stage_a_body(x_ref, pos_ref, W_in_ref, b_in_ref, W_lin_ref, W_src_ref,
                  W_dst_ref, pW1_ref, aW1_ref,
                  p1_ref, ad1_ref, as1_ref, hl_ref):
    h = jax.nn.relu(_dot(x_ref[...], W_in_ref[...]) + b_in_ref[...])
    p1_ref[...] = _dot(pos_ref[...], pW1_ref[...])
    ad1_ref[...] = _dot(h, _dot(W_dst_ref[...], aW1_ref[...]))
    as1_ref[...] = _dot(h, _dot(W_src_ref[...], aW1_ref[...]))
    hl_ref[...] = _dot(h, W_lin_ref[...])


def _stage_a(x, pos, W_in, b_in, W_lin, W_src, W_dst, pW1, aW1, n_blk):
    n = x.shape[0]
    grid = (n // n_blk,)
    row_spec = pl.BlockSpec((n_blk, C), lambda i: (i, 0))
    w_spec = pl.BlockSpec((C, C), lambda i: (0, 0))
    b_spec = pl.BlockSpec((1, C), lambda i: (0, 0))
    out_sh = jax.ShapeDtypeStruct((n, C), f32)
    return pl.pallas_call(
        _stage_a_body,
        grid=grid,
        in_specs=[row_spec, row_spec, w_spec, b_spec, w_spec, w_spec, w_spec,
                  w_spec, w_spec],
        out_specs=[row_spec, row_spec, row_spec, row_spec],
        out_shape=[out_sh, out_sh, out_sh, out_sh],
        compiler_params=pltpu.CompilerParams(
            dimension_semantics=("arbitrary",)),
    )(x, pos, W_in, b_in.reshape(1, C), W_lin, W_src, W_dst, pW1, aW1)


def _stage_b(p1, ad1, as1, sd, K):
    e = sd.shape[0] // 2
    n = p1.shape[0]
    ew = e // NW
    nch = ew // K
    npair = nch // 2
    mesh = plsc.VectorSubcoreMesh(core_axis_name="c", subcore_axis_name="s")
    out_sh = jax.ShapeDtypeStruct((e, C), f32)

    slot_t = [
        pltpu.VMEM((2 * K,), jnp.int32),
        pltpu.VMEM((K, C), f32),
        pltpu.VMEM((K, C), f32),
        pltpu.VMEM((K, C), f32),
        pltpu.VMEM((K, C), f32),
        pltpu.SemaphoreType.DMA,
        pltpu.SemaphoreType.DMA,
    ]

    @functools.partial(
        pl.kernel,
        out_type=[out_sh, out_sh],
        mesh=mesh,
        scratch_types=slot_t + slot_t,
    )
    def kern(p1_h, ad1_h, as1_h, sd_h, g1_h, w_h,
             sdA, bd1A, bs1A, bd2A, bs2A, gsA, wsA,
             sdB, bd1B, bs1B, bd2B, bs2B, gsB, wsB):
        cid = lax.axis_index("c")
        sid = lax.axis_index("s")
        wid = cid * NS + sid
        base = wid * ew

        slots = ((sdA, bd1A, bs1A, bd2A, bs2A, gsA, wsA),
                 (sdB, bd1B, bs1B, bd2B, bs2B, gsB, wsB))

        def issue(s, off):
            sdx, bd1, bs1, bd2, bs2, gs, _ = slots[s]
            pltpu.sync_copy(sd_h.at[pl.ds(2 * off, 2 * K)], sdx)
            six = sdx.at[pl.ds(0, K)]
            dix = sdx.at[pl.ds(K, K)]
            pltpu.async_copy(p1_h.at[dix], bd1, gs)
            pltpu.async_copy(p1_h.at[six], bs1, gs)
            pltpu.async_copy(ad1_h.at[dix], bd2, gs)
            pltpu.async_copy(as1_h.at[six], bs2, gs)

        def drain_writes(s):
            _, bd1, _, bd2, _, _, ws = slots[s]
            pltpu.make_async_copy(bd1, g1_h.at[pl.ds(0, K)], ws).wait()
            pltpu.make_async_copy(bd2, w_h.at[pl.ds(0, K)], ws).wait()

        def process(s, off):
            sdx, bd1, bs1, bd2, bs2, gs, ws = slots[s]
            six = sdx.at[pl.ds(0, K)]
            dix = sdx.at[pl.ds(K, K)]
            pltpu.make_async_copy(p1_h.at[dix], bd1, gs).wait()
            pltpu.make_async_copy(p1_h.at[six], bs1, gs).wait()
            pltpu.make_async_copy(ad1_h.at[dix], bd2, gs).wait()
            pltpu.make_async_copy(as1_h.at[six], bs2, gs).wait()

            def row(r, carry2):
                for j in range(CV):
                    sl = pl.ds(j * LANES, LANES)
                    bd1[r, sl] = bd1[r, sl] - bs1[r, sl]
                    bd2[r, sl] = bd2[r, sl] - bs2[r, sl]
                return carry2

            lax.fori_loop(0, K, row, 0)
            pltpu.async_copy(bd1, g1_h.at[pl.ds(off, K)], ws)
            pltpu.async_copy(bd2, w_h.at[pl.ds(off, K)], ws)

        issue(0, base)

        def pair(g, carry):
            offA = base + (2 * g) * K
            offB = offA + K

            @pl.when(g > 0)
            def _():
                drain_writes(1)

            issue(1, offB)
            process(0, offA)

            @pl.when(g < npair - 1)
            def _():
                drain_writes(0)
                issue(0, offA + 2 * K)

            process(1, offB)
            return carry

        lax.fori_loop(0, npair, pair, 0)
        if nch % 2 == 1:
            off = base + (nch - 1) * K
            drain_writes(0)
            issue(0, off)
            process(0, off)
        drain_writes(0)
        drain_writes(1)

    return kern(p1, ad1, as1, sd)


def _stage_c_body(g1_ref, w_ref, pW2_ref, aW1_ref, aW2_ref,
                  pb1_ref, pb2_ref, ab1_ref, ab2_ref, ex_ref, exd_ref):
    u = jax.nn.relu(g1_ref[...] + pb1_ref[...])
    delta = jax.nn.relu(_dot(u, pW2_ref[...]) + pb2_ref[...])
    v = jax.nn.relu(w_ref[...] + _dot(delta, aW1_ref[...]) + ab1_ref[...])
    alpha = jax.nn.relu(_dot(v, aW2_ref[...]) + ab2_ref[...])
    ex = jnp.exp(jnp.minimum(alpha, 80.0))
    ex_ref[...] = ex
    exd_ref[...] = ex * delta


def _stage_c(g1, w, pW2, aW1, aW2, pb1, pb2, ab1, ab2, e_blk):
    e = g1.shape[0]
    grid = (e // e_blk,)
    row_spec = pl.BlockSpec((e_blk, C), lambda i: (i, 0))
    w_spec = pl.BlockSpec((C, C), lambda i: (0, 0))
    b_spec = pl.BlockSpec((1, C), lambda i: (0, 0))
    out_sh = jax.ShapeDtypeStruct((e, C), f32)
    return pl.pallas_call(
        _stage_c_body,
        grid=grid,
        in_specs=[row_spec, row_spec, w_spec, w_spec, w_spec,
                  b_spec, b_spec, b_spec, b_spec],
        out_specs=[row_spec, row_spec],
        out_shape=[out_sh, out_sh],
        compiler_params=pltpu.CompilerParams(
            dimension_semantics=("arbitrary",)),
    )(g1, w, pW2, aW1, aW2, pb1.reshape(1, C), pb2.reshape(1, C),
      ab1.reshape(1, C), ab2.reshape(1, C))


def _stage_d_den(ex, dst, n, K):
    e = ex.shape[0]
    ew = e // NW
    nch = ew // K
    npair = nch // 2
    npad = ((n + NS * K - 1) // (NS * K)) * (NS * K)
    zr = npad // NS
    nzc = zr // K

    mesh = plsc.VectorSubcoreMesh(core_axis_name="c", subcore_axis_name="s")

    slot_t = [
        pltpu.VMEM((K,), jnp.int32),
        pltpu.VMEM((K, C), f32),
        pltpu.SemaphoreType.DMA,
        pltpu.SemaphoreType.DMA,
    ]

    @functools.partial(
        pl.kernel,
        out_type=jax.ShapeDtypeStruct((NC, npad, C), f32),
        mesh=mesh,
        scratch_types=slot_t + slot_t + [pltpu.VMEM_SHARED((npad, C), f32)],
    )
    def kern(ex_h, dst_h, den_h,
             dixA, exbA, gsA, ssA, dixB, exbB, gsB, ssB, den_sh):
        cid = lax.axis_index("c")
        sid = lax.axis_index("s")
        wid = cid * NS + sid
        base = wid * ew
        slots = ((dixA, exbA, gsA, ssA), (dixB, exbB, gsB, ssB))

        def zrow(r, carry):
            for j in range(CV):
                exbA[r, pl.ds(j * LANES, LANES)] = jnp.zeros((LANES,), f32)
            return carry

        lax.fori_loop(0, K, zrow, 0)

        def zcopy(k, carry):
            pltpu.sync_copy(exbA, den_sh.at[pl.ds(sid * zr + k * K, K)])
            return carry

        lax.fori_loop(0, nzc, zcopy, 0)
        plsc.subcore_barrier()

        def issue(s, off):
            dix, exb, gs, _ = slots[s]
            pltpu.sync_copy(dst_h.at[pl.ds(off, K)], dix)
            pltpu.async_copy(ex_h.at[pl.ds(off, K)], exb, gs)

        def drain_scatter(s):
            dix, exb, _, ss = slots[s]
            pltpu.make_async_copy(exb, den_sh.at[dix], ss).wait()

        def process(s, off):
            dix, exb, gs, ss = slots[s]
            pltpu.make_async_copy(ex_h.at[pl.ds(off, K)], exb, gs).wait()
            pltpu.async_copy(exb, den_sh.at[dix], ss, add=True)

        issue(0, base)

        def pair(g, carry):
            offA = base + (2 * g) * K
            offB = offA + K

            @pl.when(g > 0)
            def _():
                drain_scatter(1)

            issue(1, offB)
            process(0, offA)

            @pl.when(g < npair - 1)
            def _():
                drain_scatter(0)
                issue(0, offA + 2 * K)

            process(1, offB)
            return carry

        lax.fori_loop(0, npair, pair, 0)
        if nch % 2 == 1:
            off = base + (nch - 1) * K
            drain_scatter(0)
            issue(0, off)
            process(0, off)
        drain_scatter(0)
        drain_scatter(1)
        plsc.subcore_barrier()

        def wcopy(k, carry):
            pltpu.sync_copy(den_sh.at[pl.ds(sid * zr + k * K, K)], exbA)
            pltpu.sync_copy(exbA, den_h.at[cid, pl.ds(sid * zr + k * K, K)])
            return carry

        lax.fori_loop(0, nzc, wcopy, 0)

    return kern(ex, dst)


def _stage_d_num(ex, exd, hl, src, dst, n, K):
    e = ex.shape[0]
    ew = e // NW
    nch = ew // K
    npair = nch // 2
    npad = ((n + NS * K - 1) // (NS * K)) * (NS * K)
    zr = npad // NS
    nzc = zr // K

    mesh = plsc.VectorSubcoreMesh(core_axis_name="c", subcore_axis_name="s")

    slot_t = [
        pltpu.VMEM((K,), jnp.int32),
        pltpu.VMEM((K,), jnp.int32),
        pltpu.VMEM((K, C), f32),
        pltpu.VMEM((K, C), f32),
        pltpu.VMEM((K, C), f32),
        pltpu.SemaphoreType.DMA,
        pltpu.SemaphoreType.DMA,
    ]

    @functools.partial(
        pl.kernel,
        out_type=jax.ShapeDtypeStruct((NC, npad, C), f32),
        mesh=mesh,
        scratch_types=slot_t + slot_t + [pltpu.VMEM_SHARED((npad, C), f32)],
    )
    def kern(ex_h, exd_h, hl_h, src_h, dst_h, num_h,
             sixA, dixA, exbA, exdbA, hlbA, gsA, ssA,
             sixB, dixB, exbB, exdbB, hlbB, gsB, ssB, num_sh):
        cid = lax.axis_index("c")
        sid = lax.axis_index("s")
        wid = cid * NS + sid
        base = wid * ew
        slots = ((sixA, dixA, exbA, exdbA, hlbA, gsA, ssA),
                 (sixB, dixB, exbB, exdbB, hlbB, gsB, ssB))

        def zrow(r, carry):
            for j in range(CV):
                exbA[r, pl.ds(j * LANES, LANES)] = jnp.zeros((LANES,), f32)
            return carry

        lax.fori_loop(0, K, zrow, 0)

        def zcopy(k, carry):
            pltpu.sync_copy(exbA, num_sh.at[pl.ds(sid * zr + k * K, K)])
            return carry

        lax.fori_loop(0, nzc, zcopy, 0)
        plsc.subcore_barrier()

        def issue(s, off):
            six, dix, exb, exdb, hlb, gs, _ = slots[s]
            pltpu.sync_copy(src_h.at[pl.ds(off, K)], six)
            pltpu.sync_copy(dst_h.at[pl.ds(off, K)], dix)
            pltpu.async_copy(hl_h.at[six], hlb, gs)
            pltpu.async_copy(ex_h.at[pl.ds(off, K)], exb, gs)
            pltpu.async_copy(exd_h.at[pl.ds(off, K)], exdb, gs)

        def drain_scatter(s):
            _, dix, _, exdb, _, _, ss = slots[s]
            pltpu.make_async_copy(exdb, num_sh.at[dix], ss).wait()

        def process(s, off):
            six, dix, exb, exdb, hlb, gs, ss = slots[s]
            pltpu.make_async_copy(hl_h.at[six], hlb, gs).wait()
            pltpu.make_async_copy(ex_h.at[pl.ds(off, K)], exb, gs).wait()
            pltpu.make_async_copy(exd_h.at[pl.ds(off, K)], exdb, gs).wait()

            def row(r, carry2):
                for j in range(CV):
                    sl = pl.ds(j * LANES, LANES)
                    exdb[r, sl] = exdb[r, sl] + exb[r, sl] * hlb[r, sl]
                return carry2

            lax.fori_loop(0, K, row, 0)
            pltpu.async_copy(exdb, num_sh.at[dix], ss, add=True)

        issue(0, base)

        def pair(g, carry):
            offA = base + (2 * g) * K
            offB = offA + K

            @pl.when(g > 0)
            def _():
                drain_scatter(1)

            issue(1, offB)
            process(0, offA)

            @pl.when(g < npair - 1)
            def _():
                drain_scatter(0)
                issue(0, offA + 2 * K)

            process(1, offB)
            return carry

        lax.fori_loop(0, npair, pair, 0)
        if nch % 2 == 1:
            off = base + (nch - 1) * K
            drain_scatter(0)
            issue(0, off)
            process(0, off)
        drain_scatter(0)
        drain_scatter(1)
        plsc.subcore_barrier()

        def wcopy(k, carry):
            pltpu.sync_copy(num_sh.at[pl.ds(sid * zr + k * K, K)], exbA)
            pltpu.sync_copy(exbA, num_h.at[cid, pl.ds(sid * zr + k * K, K)])
            return carry

        lax.fori_loop(0, nzc, wcopy, 0)

    return kern(ex, exd, hl, src, dst)


def _stage_e_body(num0_ref, num1_ref, den0_ref, den1_ref,
                  W_out_ref, b_out_ref, out_ref):
    num = (num0_ref[0] + num0_ref[1]) + (num1_ref[0] + num1_ref[1])
    den = (den0_ref[0] + den0_ref[1]) + (den1_ref[0] + den1_ref[1])
    r = num / (den + 1e-16)
    out_ref[...] = jax.nn.relu(_dot(r, W_out_ref[...]) + b_out_ref[...])


def _stage_e(num0, num1, den0, den1, W_out, b_out, n, n_blk):
    grid = (n // n_blk,)
    part_spec = pl.BlockSpec((NC, n_blk, C), lambda i: (0, i, 0))
    w_spec = pl.BlockSpec((C, C), lambda i: (0, 0))
    b_spec = pl.BlockSpec((1, C), lambda i: (0, 0))
    return pl.pallas_call(
        _stage_e_body,
        grid=grid,
        in_specs=[part_spec, part_spec, part_spec, part_spec, w_spec, b_spec],
        out_specs=pl.BlockSpec((n_blk, C), lambda i: (i, 0)),
        out_shape=jax.ShapeDtypeStruct((n, C), f32),
        compiler_params=pltpu.CompilerParams(
            dimension_semantics=("arbitrary",)),
    )(num0, num1, den0, den1, W_out, b_out.reshape(1, C))


def kernel(x, pos, edge_index, W_in, b_in, W_out, b_out, W_lin, W_src, W_dst,
           pW1, pb1, pW2, pb2, aW1, ab1, aW2, ab2):
    n = x.shape[0]
    src = edge_index[0]
    dst = edge_index[1]

    KB = 80
    KN = 40
    n_blk = 1000
    e_blk = 2000
    e = src.shape[0]
    e0 = (e * 3) // 5
    parts = []
    for lo, hi in ((0, e0), (e0, e)):
        s_h = src[lo:hi]
        d_h = dst[lo:hi]
        sd_h = jnp.stack([s_h.reshape(-1, KB), d_h.reshape(-1, KB)],
                         axis=1).reshape(-1)
        parts.append((s_h, d_h, sd_h))

    p1, ad1, as1, hl = _stage_a(x, pos, W_in, b_in, W_lin, W_src, W_dst,
                                pW1, aW1, n_blk)
    g1_0, w_0 = _stage_b(p1, ad1, as1, parts[0][2], KB)
    g1_1, w_1 = _stage_b(p1, ad1, as1, parts[1][2], KB)
    ex0, exd0 = _stage_c(g1_0, w_0, pW2, aW1, aW2, pb1, pb2, ab1, ab2, e_blk)
    den0 = _stage_d_den(ex0, parts[0][1], n, KB)
    num0 = _stage_d_num(ex0, exd0, hl, parts[0][0], parts[0][1], n, KN)
    ex1, exd1 = _stage_c(g1_1, w_1, pW2, aW1, aW2, pb1, pb2, ab1, ab2, e_blk)
    den1 = _stage_d_den(ex1, parts[1][1], n, KB)
    num1 = _stage_d_num(ex1, exd1, hl, parts[1][0], parts[1][1], n, KN)
    return _stage_e(num0, num1, den0, den1, W_out, b_out, n, n_blk)
